# baseline (speedup 1.0000x reference)
import jax
import jax.numpy as jnp
from jax import lax
from jax.experimental import pallas as pl
from jax.experimental.pallas import tpu as pltpu
from jax.experimental.pallas import tpu_sc as plsc

_B = 16384
_D = 64
_NC = 2
_NS = 16
_NW = _NC * _NS
_BPW = _B // _NW
_L = 16


def _body(user_h, pos_h, neg_h, ut_h, it_h, pos_out, neg_out,
          uidx, pidx, nidx, urows, prows, nrows, psc, nsc,
          sem_u, sem_p, sem_n):
    wid = lax.axis_index("s") * _NC + lax.axis_index("c")
    base = wid * _BPW

    pltpu.sync_copy(user_h.at[pl.ds(base, _BPW)], uidx)
    pltpu.sync_copy(pos_h.at[pl.ds(base, _BPW)], pidx)
    pltpu.sync_copy(neg_h.at[pl.ds(base, _BPW)], nidx)

    cu = pltpu.async_copy(ut_h.at[uidx], urows, sem_u)
    cp = pltpu.async_copy(it_h.at[pidx], prows, sem_p)
    cn = pltpu.async_copy(it_h.at[nidx], nrows, sem_n)
    cu.wait()
    cp.wait()
    cn.wait()

    lane = lax.iota(jnp.int32, _L)

    def group(g, carry):
        rows = lane + g * _L
        accp = jnp.zeros((_L,), jnp.float32)
        accn = jnp.zeros((_L,), jnp.float32)
        for d in range(_D):
            col = jnp.full((_L,), d, jnp.int32)
            u = plsc.load_gather(urows, [rows, col])
            p = plsc.load_gather(prows, [rows, col])
            n = plsc.load_gather(nrows, [rows, col])
            accp = accp + u * p
            accn = accn + u * n
        psc[pl.ds(g * _L, _L)] = accp
        nsc[pl.ds(g * _L, _L)] = accn
        return carry

    lax.fori_loop(0, _BPW // _L, group, 0)

    pltpu.sync_copy(psc, pos_out.at[pl.ds(base, _BPW)])
    pltpu.sync_copy(nsc, neg_out.at[pl.ds(base, _BPW)])


@jax.jit
def kernel(user, pos_item, neg_item, user_table, item_table):
    f32 = jnp.float32
    run = pl.kernel(
        _body,
        out_type=[jax.ShapeDtypeStruct((_B,), f32),
                  jax.ShapeDtypeStruct((_B,), f32)],
        mesh=plsc.VectorSubcoreMesh(core_axis_name="c", subcore_axis_name="s"),
        compiler_params=pltpu.CompilerParams(needs_layout_passes=False,
                                             use_tc_tiling_on_sc=False),
        scratch_types=[
            pltpu.VMEM((_BPW,), jnp.int32),
            pltpu.VMEM((_BPW,), jnp.int32),
            pltpu.VMEM((_BPW,), jnp.int32),
            pltpu.VMEM((_BPW, _D), f32),
            pltpu.VMEM((_BPW, _D), f32),
            pltpu.VMEM((_BPW, _D), f32),
            pltpu.VMEM((_BPW,), f32),
            pltpu.VMEM((_BPW,), f32),
            pltpu.SemaphoreType.DMA,
            pltpu.SemaphoreType.DMA,
            pltpu.SemaphoreType.DMA,
        ],
    )
    pos_score, neg_score = run(user.astype(jnp.int32),
                               pos_item.astype(jnp.int32),
                               neg_item.astype(jnp.int32),
                               user_table, item_table)
    return (pos_score, neg_score)


# speedup vs baseline: 1.2880x; 1.2880x over previous
import jax
import jax.numpy as jnp
from jax import lax
from jax.experimental import pallas as pl
from jax.experimental.pallas import tpu as pltpu
from jax.experimental.pallas import tpu_sc as plsc

_B = 16384
_D = 64
_NC = 2
_NS = 16
_NW = _NC * _NS
_BPW = _B // _NW
_CHUNK = 128
_NCHUNK = _BPW // _CHUNK
_L = 16
_GPC = _CHUNK // _L


def _rot(x, k):
    perm = (lax.iota(jnp.int32, _L) + k) & (_L - 1)
    return jnp.take_along_axis(x, perm, axis=0, mode="promise_in_bounds")


def _merge(a, b, block):
    half = block // 2
    first = (lax.iota(jnp.int32, _L) % block) < half
    m1 = jnp.where(first, a, _rot(b, -half))
    m2 = jnp.where(first, _rot(a, half), b)
    return m1 + m2


def _lane_sums(v):
    y = [_merge(v[i], v[i + 8], 16) for i in range(8)]
    z = [_merge(y[i], y[i + 4], 8) for i in range(4)]
    w = [_merge(z[i], z[i + 2], 4) for i in range(2)]
    return _merge(w[0], w[1], 2)


def _body(user_h, pos_h, neg_h, ut_h, it_h, pos_out, neg_out,
          uidx, pidx, nidx, uq, pq, nq, ubuf, pbuf, nbuf, psc, nsc,
          *sems):
    wid = lax.axis_index("s") * _NC + lax.axis_index("c")
    base = wid * _BPW

    pltpu.sync_copy(user_h.at[pl.ds(base, _BPW)], uidx)
    pltpu.sync_copy(pos_h.at[pl.ds(base, _BPW)], pidx)
    pltpu.sync_copy(neg_h.at[pl.ds(base, _BPW)], nidx)

    def halve(i, carry):
        s = pl.ds(i * _L, _L)
        uq[s] = uidx[s] >> 1
        pq[s] = pidx[s] >> 1
        nq[s] = nidx[s] >> 1
        return carry

    lax.fori_loop(0, _BPW // _L, halve, 0)

    def fire(c, buf_slot):
        s = pl.ds(c * _CHUNK, _CHUNK)
        return (
            pltpu.async_copy(ut_h.at[uq.at[s]], ubuf.at[buf_slot],
                             sems[buf_slot]),
            pltpu.async_copy(it_h.at[pq.at[s]], pbuf.at[buf_slot],
                             sems[2 + buf_slot]),
            pltpu.async_copy(it_h.at[nq.at[s]], nbuf.at[buf_slot],
                             sems[4 + buf_slot]),
        )

    def compute_chunk(c, buf_slot):
        ub, pb, nb = ubuf.at[buf_slot], pbuf.at[buf_slot], nbuf.at[buf_slot]

        def group(g, carry):
            gabs = c * _GPC + g
            gs = pl.ds(gabs * _L, _L)
            offu = (uidx[gs] & 1) << 6
            offp = (pidx[gs] & 1) << 6
            offn = (nidx[gs] & 1) << 6
            sp, sn = [], []
            for r in range(_L):
                slot = g * _L + r
                ou, op, on = offu[r], offp[r], offn[r]
                accp = None
                accn = None
                for k in range(4):
                    du = pl.multiple_of(ou + k * _L, _L)
                    dp = pl.multiple_of(op + k * _L, _L)
                    dn = pl.multiple_of(on + k * _L, _L)
                    u = ub[slot, pl.ds(du, _L)]
                    p = pb[slot, pl.ds(dp, _L)]
                    n = nb[slot, pl.ds(dn, _L)]
                    accp = u * p if accp is None else accp + u * p
                    accn = u * n if accn is None else accn + u * n
                sp.append(accp)
                sn.append(accn)
            psc[gs] = _lane_sums(sp)
            nsc[gs] = _lane_sums(sn)
            return carry

        lax.fori_loop(0, _GPC, group, 0)

    copies = [None] * _NCHUNK
    copies[0] = fire(0, 0)
    copies[1] = fire(1, 1)
    for c in range(_NCHUNK):
        for cp in copies[c]:
            cp.wait()
        compute_chunk(c, c % 2)
        if c + 2 < _NCHUNK:
            copies[c + 2] = fire(c + 2, c % 2)

    pltpu.sync_copy(psc, pos_out.at[pl.ds(base, _BPW)])
    pltpu.sync_copy(nsc, neg_out.at[pl.ds(base, _BPW)])


@jax.jit
def kernel(user, pos_item, neg_item, user_table, item_table):
    f32 = jnp.float32
    nu, dim = user_table.shape
    ni, _ = item_table.shape
    ut2 = user_table.reshape(nu // 2, 2 * dim)
    it2 = item_table.reshape(ni // 2, 2 * dim)
    run = pl.kernel(
        _body,
        out_type=[jax.ShapeDtypeStruct((_B,), f32),
                  jax.ShapeDtypeStruct((_B,), f32)],
        mesh=plsc.VectorSubcoreMesh(core_axis_name="c", subcore_axis_name="s"),
        compiler_params=pltpu.CompilerParams(needs_layout_passes=False),
        scratch_types=[
            pltpu.VMEM((_BPW,), jnp.int32),
            pltpu.VMEM((_BPW,), jnp.int32),
            pltpu.VMEM((_BPW,), jnp.int32),
            pltpu.VMEM((_BPW,), jnp.int32),
            pltpu.VMEM((_BPW,), jnp.int32),
            pltpu.VMEM((_BPW,), jnp.int32),
            pltpu.VMEM((2, _CHUNK, 2 * _D), f32),
            pltpu.VMEM((2, _CHUNK, 2 * _D), f32),
            pltpu.VMEM((2, _CHUNK, 2 * _D), f32),
            pltpu.VMEM((_BPW,), f32),
            pltpu.VMEM((_BPW,), f32),
        ] + [pltpu.SemaphoreType.DMA] * 6,
    )
    pos_score, neg_score = run(user.astype(jnp.int32),
                               pos_item.astype(jnp.int32),
                               neg_item.astype(jnp.int32),
                               ut2, it2)
    return (pos_score, neg_score)
